# baseline (speedup 1.0000x reference)
import jax
import jax.numpy as jnp
from jax.experimental import pallas as pl
from jax.experimental.pallas import tpu as pltpu

_SPC = 4
_BS = 512
_D = 128
_NCOL = _BS - _SPC + 1
_T = 0.07
_N_LEN = 100000.0
_RB = 128
_NBLK = _BS // _RB


def _nce_kernel(x_ref, outs_ref, probs_ref, e_ref, zsum_ref, psum_ref):
    k = pl.program_id(0)

    @pl.when(k < _NBLK)
    def _compute():
        x = x_ref[:, :]
        xb = x_ref[pl.ds(k * _RB, _RB), :]
        g = jax.lax.dot_general(xb, x, (((1,), (1,)), ((), ())),
                                preferred_element_type=jnp.float32)

        row = jax.lax.broadcasted_iota(jnp.int32, (_RB, _BS), 0) + k * _RB
        col = jax.lax.broadcasted_iota(jnp.int32, (_RB, _BS), 1)

        in_group = (col // _SPC) == (row // _SPC)
        off_diag = col != row
        pos_sum = jnp.sum(jnp.where(in_group & off_diag, g, 0.0), axis=1,
                          keepdims=True)
        pos_logit = pos_sum * (1.0 / (_SPC - 1))

        a = g[:, : _BS - _SPC]
        b = g[:, _SPC:]
        kk = jax.lax.broadcasted_iota(jnp.int32, (_RB, _BS - _SPC), 1)
        rg = (jax.lax.broadcasted_iota(jnp.int32, (_RB, _BS - _SPC), 0)
              + k * _RB) // _SPC
        neg = jnp.where(kk < _SPC * rg, a, b)

        logits = jnp.concatenate([pos_logit, neg], axis=1)
        e = jnp.exp(logits * (1.0 / _T))
        e_ref[pl.ds(k * _RB, _RB), :] = e
        bsum = jnp.sum(e, keepdims=True).reshape(1, 1)

        @pl.when(k == 0)
        def _():
            zsum_ref[:, :] = bsum

        @pl.when(k > 0)
        def _():
            zsum_ref[:, :] = zsum_ref[:, :] + bsum

    @pl.when(k >= _NBLK)
    def _emit():
        b = k - _NBLK
        z = (zsum_ref[:, :] * (1.0 / (_BS * _NCOL))) * _N_LEN
        outs = e_ref[pl.ds(b * _RB, _RB), :] / z
        outs_ref[:, :] = outs
        rowsum = jnp.sum(outs, axis=1, keepdims=True)
        pm0 = outs[:, 0:1] / rowsum
        bp = jnp.sum(pm0, keepdims=True).reshape(1, 1)

        @pl.when(b == 0)
        def _():
            psum_ref[:, :] = bp

        @pl.when(b > 0)
        def _():
            psum_ref[:, :] = psum_ref[:, :] + bp

        @pl.when(b == _NBLK - 1)
        def _():
            probs_ref[:, :] = psum_ref[:, :] * (1.0 / _BS)


def kernel(x, i):
    del i
    outs, probs = pl.pallas_call(
        _nce_kernel,
        grid=(2 * _NBLK,),
        in_specs=[pl.BlockSpec((_BS, _D), lambda k: (0, 0))],
        out_specs=(
            pl.BlockSpec((_RB, _NCOL), lambda k: (jnp.maximum(k - _NBLK, 0), 0)),
            pl.BlockSpec((1, 1), lambda k: (0, 0)),
        ),
        out_shape=(
            jax.ShapeDtypeStruct((_BS, _NCOL), jnp.float32),
            jax.ShapeDtypeStruct((1, 1), jnp.float32),
        ),
        scratch_shapes=[
            pltpu.VMEM((_BS, _NCOL), jnp.float32),
            pltpu.VMEM((1, 1), jnp.float32),
            pltpu.VMEM((1, 1), jnp.float32),
        ],
        compiler_params=pltpu.CompilerParams(
            dimension_semantics=("arbitrary",)),
    )(x)
    return outs, probs.reshape(())


# speedup vs baseline: 1.9776x; 1.9776x over previous
import jax
import jax.numpy as jnp
from jax.experimental import pallas as pl
from jax.experimental.pallas import tpu as pltpu

_SPC = 4
_BS = 512
_D = 128
_NCOL = _BS - _SPC + 1
_T = 0.07
_N_LEN = 100000.0


def _nce_kernel(x_ref, outs_ref, probs_ref):
    x = x_ref[:, :]
    u = jnp.roll(x, 1, axis=0)
    v = jnp.roll(x, -3, axis=0)
    dn = (((1,), (1,)), ((), ()))
    a = jax.lax.dot_general(x, u, dn,
                            preferred_element_type=jnp.float32)
    b = jax.lax.dot_general(x, v, dn,
                            preferred_element_type=jnp.float32)

    row = jax.lax.broadcasted_iota(jnp.int32, (_BS, _BS), 0)
    col = jax.lax.broadcasted_iota(jnp.int32, (_BS, _BS), 1)
    rg = row // _SPC

    gcol = (col + (_BS - 1)) % _BS
    in_group = (gcol // _SPC) == rg
    off_diag = gcol != row
    pos_sum = jnp.sum(jnp.where(in_group & off_diag, a, 0.0), axis=1,
                      keepdims=True)
    pos_logit = pos_sum * (1.0 / (_SPC - 1))

    sel = jnp.where(col <= _SPC * rg, a, b)
    logits = jnp.where(col == 0, pos_logit,
                       jnp.where(col < _NCOL, sel, -jnp.inf))

    e = jnp.exp(logits * (1.0 / _T))
    z = (jnp.sum(e) * (1.0 / (_BS * _NCOL))) * _N_LEN
    outs = e / z
    outs_ref[:, :] = outs

    rowsum = jnp.sum(outs, axis=1, keepdims=True)
    pm0 = outs[:, 0:1] / rowsum
    probs_ref[:, :] = jnp.sum(pm0, axis=0, keepdims=True) * (1.0 / _BS)


def kernel(x, i):
    del i
    outs_pad, probs = pl.pallas_call(
        _nce_kernel,
        out_shape=(
            jax.ShapeDtypeStruct((_BS, _BS), jnp.float32),
            jax.ShapeDtypeStruct((1, 1), jnp.float32),
        ),
    )(x)
    return outs_pad[:, :_NCOL], probs.reshape(())


# speedup vs baseline: 1.9934x; 1.0079x over previous
import jax
import jax.numpy as jnp
from jax.experimental import pallas as pl
from jax.experimental.pallas import tpu as pltpu

_SPC = 4
_BS = 512
_D = 128
_NCOL = _BS - _SPC + 1
_T = 0.07
_N_LEN = 100000.0


def _nce_kernel(x_ref, outs_ref, probs_ref):
    x = x_ref[:, :]
    u = jnp.roll(x, 1, axis=0)
    v = jnp.roll(x, -3, axis=0)
    dn = (((1,), (1,)), ((), ()))
    a = jax.lax.dot_general(x, u, dn,
                            preferred_element_type=jnp.float32)
    b = jax.lax.dot_general(x, v, dn,
                            preferred_element_type=jnp.float32)

    row = jax.lax.broadcasted_iota(jnp.int32, (_BS, _BS), 0)
    col = jax.lax.broadcasted_iota(jnp.int32, (_BS, _BS), 1)
    rg = row // _SPC

    gcol = (col + (_BS - 1)) % _BS
    in_group = (gcol // _SPC) == rg
    off_diag = gcol != row
    pos_sum = jnp.sum(jnp.where(in_group & off_diag, a, 0.0), axis=1,
                      keepdims=True)
    pos_logit = pos_sum * (1.0 / (_SPC - 1))

    sel = jnp.where(col <= _SPC * rg, a, b)
    logits = jnp.where(col == 0, pos_logit,
                       jnp.where(col < _NCOL, sel, -jnp.inf))

    e = jnp.exp(logits * (1.0 / _T))
    z = (jnp.sum(e) * (1.0 / (_BS * _NCOL))) * _N_LEN
    outs = e / z
    outs_ref[:, :] = outs[:, :_NCOL]

    rowsum = jnp.sum(outs, axis=1, keepdims=True)
    pm0 = outs[:, 0:1] / rowsum
    probs_ref[:, :] = jnp.sum(pm0, axis=0, keepdims=True) * (1.0 / _BS)


def kernel(x, i):
    del i
    outs, probs = pl.pallas_call(
        _nce_kernel,
        out_shape=(
            jax.ShapeDtypeStruct((_BS, _NCOL), jnp.float32),
            jax.ShapeDtypeStruct((1, 1), jnp.float32),
        ),
    )(x)
    return outs, probs.reshape(())
